# baseline (speedup 1.0000x reference)
import functools

import jax
import jax.numpy as jnp
from jax import lax
from jax.experimental import pallas as pl
from jax.experimental.pallas import tpu as pltpu
from jax.experimental.pallas import tpu_sc as plsc

N = 10000
NP = 10240
E = 160000
D = 256
DH = D // 2
NSC = 16
EP = NSC * NP
EPS = EP // NSC
K = 64
NB = EPS // K
NW = 20
R = 4
RPS = NP // NSC
RB = 1000


def _stage1_body(x_ref, w_ref, b_ref, c_ref, t0_ref, t1_ref):
    x = x_ref[...]
    c = c_ref[0, 0]
    sq = jnp.sqrt(c)
    nrm = jnp.sqrt(jnp.sum(x * x, axis=1, keepdims=True))
    z = sq * nrm
    atz = 0.5 * jnp.log((1.0 + z) / (1.0 - z))
    tang = x * (2.0 / sq * atz / nrm)
    res = lax.dot_general(tang, w_ref[...], (((1,), (1,)), ((), ())),
                          preferred_element_type=jnp.float32) + b_ref[...]
    t0_ref[...] = res[:, :DH]
    t1_ref[...] = res[:, DH:]


def _transform(x, w, b2, c2):
    return pl.pallas_call(
        _stage1_body,
        grid=(N // RB,),
        in_specs=[
            pl.BlockSpec((RB, D), lambda i: (i, 0)),
            pl.BlockSpec((D, D), lambda i: (0, 0)),
            pl.BlockSpec((1, D), lambda i: (0, 0)),
            pl.BlockSpec(memory_space=pltpu.SMEM),
        ],
        out_specs=[
            pl.BlockSpec((RB, DH), lambda i: (i, 0)),
            pl.BlockSpec((RB, DH), lambda i: (i, 0)),
        ],
        out_shape=[
            jax.ShapeDtypeStruct((N, DH), jnp.float32),
            jax.ShapeDtypeStruct((N, DH), jnp.float32),
        ],
    )(x, w, b2, c2)


def _sc_body(t0_hbm, t1_hbm, src_hbm, dst_hbm, s0_hbm, s1_hbm, cnt_hbm,
             acc, cacc, sidx, didx, r0, r1, r2, r3, ones, zcb,
             g0, g1, g2, g3, ss0, ss1, ss2, ss3, semc):
    cid = lax.axis_index("c")
    sid = lax.axis_index("s")
    base = sid * RPS
    bufs = (r0, r1, r2, r3)
    gsems = (g0, g1, g2, g3)
    ssems = (ss0, ss1, ss2, ss3)

    def fz(i, _):
        for j in range(DH // 16):
            r0[i, pl.ds(j * 16, 16)] = jnp.zeros((16,), jnp.float32)
        zcb[i] = jnp.zeros((16,), jnp.float32)
        ones[i] = jnp.ones((16,), jnp.float32)
        return 0
    lax.fori_loop(0, K, fz, 0)

    for k in range(RPS // K):
        pltpu.sync_copy(r0, acc.at[pl.ds(base + k * K, K)])

    @pl.when(cid == 0)
    def _():
        for k in range(RPS // K):
            pltpu.sync_copy(zcb, cacc.at[pl.ds(base + k * K, K)])

    plsc.subcore_barrier()

    def do_edges(t_hbm, with_cnt):
        def gather(i, r):
            pltpu.async_copy(t_hbm.at[sidx.at[i]], bufs[r], gsems[r])

        def gwait(r):
            pltpu.make_async_copy(t_hbm.at[pl.ds(0, K)], bufs[r], gsems[r]).wait()

        def scatter(i, r):
            pltpu.async_copy(bufs[r], acc.at[didx.at[i]], ssems[r], add=True)
            if with_cnt:
                pltpu.async_copy(ones, cacc.at[didx.at[i]], semc, add=True)

        def swait(r):
            pltpu.make_async_copy(bufs[r], acc.at[pl.ds(0, K)], ssems[r]).wait()

        def window(w, _):
            pltpu.sync_copy(src_hbm.at[sid, pl.ds(w * NW, NW)], sidx)
            pltpu.sync_copy(dst_hbm.at[sid, pl.ds(w * NW, NW)], didx)
            for r in range(R):
                gather(r, r)

            def body(j, _):
                i0 = R * j
                for r in range(R):
                    gwait(r)
                    scatter(i0 + r, r)
                for r in range(R):
                    swait(r)
                    gather(i0 + R + r, r)
                return 0
            lax.fori_loop(0, NW // R - 1, body, 0)
            for r in range(R):
                gwait(r)
                scatter(NW - R + r, r)
            for r in range(R):
                swait(r)
            if with_cnt:
                pltpu.make_async_copy(cnt_hbm.at[pl.ds(0, NW * K)],
                                      cacc.at[pl.ds(0, NW * K)], semc).wait()
            return 0
        lax.fori_loop(0, NB // NW, window, 0)

    @pl.when(cid == 0)
    def _():
        do_edges(t0_hbm, True)
        plsc.subcore_barrier()
        pltpu.sync_copy(acc.at[pl.ds(base, RPS)], s0_hbm.at[pl.ds(base, RPS)])
        pltpu.sync_copy(cacc.at[pl.ds(base, RPS)], cnt_hbm.at[pl.ds(base, RPS)])

    @pl.when(cid == 1)
    def _():
        do_edges(t1_hbm, False)
        plsc.subcore_barrier()
        pltpu.sync_copy(acc.at[pl.ds(base, RPS)], s1_hbm.at[pl.ds(base, RPS)])


def _scatter_mean(t0, t1, src3, dst3):
    mesh = plsc.VectorSubcoreMesh(core_axis_name="c", subcore_axis_name="s")
    f = pl.kernel(
        _sc_body,
        out_type=[
            jax.ShapeDtypeStruct((NP, DH), jnp.float32),
            jax.ShapeDtypeStruct((NP, DH), jnp.float32),
            jax.ShapeDtypeStruct((NP, 16), jnp.float32),
        ],
        mesh=mesh,
        scratch_types=[
            pltpu.VMEM_SHARED((NP, DH), jnp.float32),
            pltpu.VMEM_SHARED((NP, 16), jnp.float32),
            pltpu.VMEM((NW, K), jnp.int32),
            pltpu.VMEM((NW, K), jnp.int32),
            pltpu.VMEM((K, DH), jnp.float32),
            pltpu.VMEM((K, DH), jnp.float32),
            pltpu.VMEM((K, DH), jnp.float32),
            pltpu.VMEM((K, DH), jnp.float32),
            pltpu.VMEM((K, 16), jnp.float32),
            pltpu.VMEM((K, 16), jnp.float32),
            pltpu.SemaphoreType.DMA,
            pltpu.SemaphoreType.DMA,
            pltpu.SemaphoreType.DMA,
            pltpu.SemaphoreType.DMA,
            pltpu.SemaphoreType.DMA,
            pltpu.SemaphoreType.DMA,
            pltpu.SemaphoreType.DMA,
            pltpu.SemaphoreType.DMA,
            pltpu.SemaphoreType.DMA,
        ],
        compiler_params=pltpu.CompilerParams(use_tc_tiling_on_sc=False),
    )
    return f(t0, t1, src3, dst3)


def _stage3_body(s0_ref, s1_ref, cnt_ref, c_ref, out_ref):
    s = jnp.concatenate([s0_ref[...], s1_ref[...]], axis=1)
    cntv = cnt_ref[:, 0:1]
    neigh = s / jnp.maximum(cntv, 1.0)
    c = c_ref[0, 0]
    sq = jnp.sqrt(c)
    nv = jnp.sqrt(jnp.sum(neigh * neigh, axis=1, keepdims=True))
    out_ref[...] = jnp.tanh(sq * nv * 0.5) * neigh / (sq * nv)


def _expmap(s0, s1, cnt, c2):
    return pl.pallas_call(
        _stage3_body,
        grid=(N // RB,),
        in_specs=[
            pl.BlockSpec((RB, DH), lambda i: (i, 0)),
            pl.BlockSpec((RB, DH), lambda i: (i, 0)),
            pl.BlockSpec((RB, 16), lambda i: (i, 0)),
            pl.BlockSpec(memory_space=pltpu.SMEM),
        ],
        out_specs=pl.BlockSpec((RB, D), lambda i: (i, 0)),
        out_shape=jax.ShapeDtypeStruct((N, D), jnp.float32),
    )(s0, s1, cnt, c2)


def kernel(node_embeddings, edge_index, lin_w, lin_b, curvature):
    c2 = curvature.reshape(1, 1)
    b2 = lin_b.reshape(1, D)
    t0, t1 = _transform(node_embeddings, lin_w, b2, c2)
    pad = EP - E
    src_p = jnp.concatenate(
        [edge_index[0], jnp.zeros((pad,), jnp.int32)])
    dst_p = jnp.concatenate(
        [edge_index[1], N + (jnp.arange(pad, dtype=jnp.int32) % (NP - N))])
    src3 = src_p.reshape(NSC, NB, K)
    dst3 = dst_p.reshape(NSC, NB, K)
    s0, s1, cnt = _scatter_mean(t0, t1, src3, dst3)
    return _expmap(s0, s1, cnt, c2)
